# baseline (speedup 1.0000x reference)
import functools

import jax
import jax.numpy as jnp
from jax import lax
from jax.experimental import pallas as pl
from jax.experimental.pallas import tpu as pltpu
from jax.experimental.pallas import tpu_sc as plsc

B = 16384
D = 128
NC = 2
NS = 16
NW = NC * NS
B_PER_W = B // NW
CH = 4
RPC = B_PER_W // CH


def _build():
    mesh = plsc.VectorSubcoreMesh(core_axis_name="c", subcore_axis_name="s")

    @functools.partial(
        pl.kernel,
        mesh=mesh,
        out_type=jax.ShapeDtypeStruct((B, D), jnp.float32),
        scratch_types=[
            pltpu.VMEM((CH, RPC), jnp.int32),
            pltpu.VMEM_SHARED((4, D), jnp.float32),
            pltpu.VMEM((CH, RPC, D), jnp.float32),
            pltpu.SemaphoreType.DMA,
            pltpu.SemaphoreType.DMA,
            pltpu.SemaphoreType.DMA,
        ],
    )
    def gather_kernel(idx_hbm, table_hbm, out_hbm, idx_v, tbl_sh, rows_v,
                      ssem, gsem, osem):
        sid = lax.axis_index("s")
        wid = sid * NC + lax.axis_index("c")
        base = wid * B_PER_W
        @pl.when(sid == 0)
        def _():
            pltpu.async_copy(table_hbm, tbl_sh, gsem)

        ic = pltpu.async_copy(idx_hbm.at[pl.ds(wid * CH, CH)], idx_v, ssem)

        @pl.when(sid == 0)
        def _():
            pltpu.make_async_copy(table_hbm, tbl_sh, gsem).wait()

        ic.wait()
        plsc.subcore_barrier()
        tbl = tbl_sh

        gs = [
            pltpu.async_copy(tbl.at[idx_v.at[c]], rows_v.at[c], gsem)
            for c in range(CH)
        ]
        outs = []
        for c in range(CH):
            gs[c].wait()
            outs.append(pltpu.async_copy(
                rows_v.at[c],
                out_hbm.at[pl.ds(base + c * RPC, RPC)], osem))
        for o in outs:
            o.wait()

    return gather_kernel


_GATHER = None


def kernel(emotion_label, table):
    global _GATHER
    if _GATHER is None:
        _GATHER = _build()
    idx = emotion_label.astype(jnp.int32).reshape(NW * CH, RPC)
    return _GATHER(idx, table)


# speedup vs baseline: 1.0011x; 1.0011x over previous
import functools

import jax
import jax.numpy as jnp
from jax import lax
from jax.experimental import pallas as pl
from jax.experimental.pallas import tpu as pltpu
from jax.experimental.pallas import tpu_sc as plsc

B = 16384
D = 128
NC = 2
NS = 16
NW = NC * NS
B_PER_W = B // NW
CH = 4
RPC = B_PER_W // CH


def _build():
    mesh = plsc.VectorSubcoreMesh(core_axis_name="c", subcore_axis_name="s")

    @functools.partial(
        pl.kernel,
        mesh=mesh,
        out_type=jax.ShapeDtypeStruct((B, D), jnp.float32),
        scratch_types=[
            pltpu.VMEM((CH, RPC), jnp.int32),
            pltpu.VMEM_SHARED((4, D), jnp.float32),
            pltpu.VMEM((CH, RPC, D), jnp.float32),
            pltpu.SemaphoreType.DMA,
            pltpu.SemaphoreType.DMA,
            pltpu.SemaphoreType.DMA,
        ],
    )
    def gather_kernel(idx_hbm, table_hbm, out_hbm, idx_v, tbl_sh, rows_v,
                      ssem, gsem, osem):
        sid = lax.axis_index("s")
        wid = sid * NC + lax.axis_index("c")
        base = wid * B_PER_W
        ic = pltpu.async_copy(idx_hbm.at[pl.ds(wid * CH, CH)], idx_v, ssem)

        @pl.when(sid == 0)
        def _():
            pltpu.sync_copy(table_hbm, tbl_sh)

        ic.wait()
        plsc.subcore_barrier()
        tbl = tbl_sh

        gs = [
            pltpu.async_copy(tbl.at[idx_v.at[c]], rows_v.at[c], gsem)
            for c in range(CH)
        ]
        outs = []
        for c in range(CH):
            gs[c].wait()
            outs.append(pltpu.async_copy(
                rows_v.at[c],
                out_hbm.at[pl.ds(base + c * RPC, RPC)], osem))
        for o in outs:
            o.wait()

    return gather_kernel


_GATHER = None


def kernel(emotion_label, table):
    global _GATHER
    if _GATHER is None:
        _GATHER = _build()
    idx = emotion_label.astype(jnp.int32).reshape(NW * CH, RPC)
    return _GATHER(idx, table)
